# baseline (speedup 1.0000x reference)
import functools

import jax
import jax.numpy as jnp
from jax import lax
from jax.experimental import pallas as pl
from jax.experimental.pallas import tpu as pltpu
from jax.experimental.pallas import tpu_sc as plsc

B = 16
S = 2048
D = 1024
NC = 2
NS = 16
HALF = S // 2
R = 32
NBUF = 2
LAG = 1
NCHUNK = HALF // R


def _pos_embed_body(x_hbm, pos_hbm, out_hbm, xall_v, *scratch):
    idxs = scratch[:NBUF]
    bufs = scratch[NBUF:2 * NBUF]
    sins = scratch[2 * NBUF:3 * NBUF]
    souts = scratch[3 * NBUF:4 * NBUF]

    c = lax.axis_index("c")
    s = lax.axis_index("s")
    wid = c * NS + s
    i = wid // 2
    h = wid % 2
    j0 = h * HALF

    ia = pl.multiple_of((i // 8) * 8, 8)
    pltpu.sync_copy(x_hbm.at[pl.ds(ia, 8), :], xall_v)
    ir = i - (i // 8) * 8

    def scan_body(k, acc):
        v = xall_v[ir, pl.ds(k * 16, 16)]
        cand = jnp.where(v != 0, lax.iota(jnp.int32, 16) + k * 16, S)
        return jnp.minimum(acc, cand)

    acc = lax.fori_loop(0, S // 16, scan_body,
                        jnp.full((16,), S, jnp.int32), unroll=8)
    m = jnp.int32(S)
    for l in range(16):
        m = jnp.minimum(m, acc[l])
    fi = jnp.where(m >= S, 0, m)

    rel0 = j0 - fi

    def in_copy(t, b):
        base = rel0 + t * R
        for q in range(R // 16):
            idxs[b][pl.ds(q * 16, 16)] = jnp.clip(
                lax.iota(jnp.int32, 16) + base + q * 16, 0, S - 1)
        return pltpu.make_async_copy(pos_hbm.at[idxs[b]], bufs[b], sins[b])

    def in_wait(b):
        pltpu.make_async_copy(pos_hbm.at[idxs[b]], bufs[b], sins[b]).wait()

    def zero_fix(t, b):
        nz = jnp.clip(fi - (j0 + t * R), 0, R)

        @pl.when(nz > 0)
        def _():
            def zrow(r, _):
                for col in range(D // 16):
                    bufs[b][r, pl.ds(col * 16, 16)] = jnp.zeros(
                        (16,), jnp.float32)
                return 0
            lax.fori_loop(0, nz, zrow, 0)

    def out_copy(t, b):
        row = pl.multiple_of(i * S + j0 + t * R, 8)
        return pltpu.make_async_copy(
            bufs[b], out_hbm.at[pl.ds(row, R), :], souts[b])

    for b in range(NBUF):
        in_copy(b, b).start()

    def group(g, _):
        for b in range(NBUF):
            t = g * NBUF + b
            in_wait(b)
            zero_fix(t, b)
            out_copy(t, b).start()
            rb = (b - LAG) % NBUF
            tr = t - LAG

            @pl.when(jnp.logical_and(tr >= 0, tr + NBUF < NCHUNK))
            def _():
                out_copy(tr, rb).wait()
                in_copy(tr + NBUF, rb).start()
        return 0

    lax.fori_loop(0, NCHUNK // NBUF, group, 0)

    for b in range(NBUF):
        out_copy(NCHUNK - NBUF + b, b).wait()


_pos_embed = functools.partial(
    pl.kernel,
    out_type=jax.ShapeDtypeStruct((B * S, D), jnp.float32),
    mesh=plsc.VectorSubcoreMesh(core_axis_name="c", subcore_axis_name="s"),
    scratch_types=(
        [pltpu.VMEM((8, S), jnp.int32)]
        + [pltpu.VMEM((R,), jnp.int32) for _ in range(NBUF)]
        + [pltpu.VMEM((R, D), jnp.float32) for _ in range(NBUF)]
        + [pltpu.SemaphoreType.DMA for _ in range(2 * NBUF)]
    ),
)(_pos_embed_body)


@jax.jit
def kernel(x, pos):
    out = _pos_embed(x.astype(jnp.int32), pos)
    return out.reshape(B, S, D)


# speedup vs baseline: 1.0364x; 1.0364x over previous
import functools

import jax
import jax.numpy as jnp
from jax import lax
from jax.experimental import pallas as pl
from jax.experimental.pallas import tpu as pltpu
from jax.experimental.pallas import tpu_sc as plsc

B = 16
S = 2048
D = 1024
NC = 2
NS = 16
HALF = S // 2
R = 16
NBUF = 4
LAG = 2
NCHUNK = HALF // R


def _pos_embed_body(x_hbm, pos_hbm, out_hbm, xall_v, *scratch):
    idxs = scratch[:NBUF]
    bufs = scratch[NBUF:2 * NBUF]
    sins = scratch[2 * NBUF:3 * NBUF]
    souts = scratch[3 * NBUF:4 * NBUF]

    c = lax.axis_index("c")
    s = lax.axis_index("s")
    wid = c * NS + s
    i = wid // 2
    h = wid % 2
    j0 = h * HALF

    ia = pl.multiple_of((i // 8) * 8, 8)
    pltpu.sync_copy(x_hbm.at[pl.ds(ia, 8), :], xall_v)
    ir = i - (i // 8) * 8

    def scan_body(k, acc):
        v = xall_v[ir, pl.ds(k * 16, 16)]
        cand = jnp.where(v != 0, lax.iota(jnp.int32, 16) + k * 16, S)
        return jnp.minimum(acc, cand)

    acc = lax.fori_loop(0, S // 16, scan_body,
                        jnp.full((16,), S, jnp.int32), unroll=8)
    m = jnp.int32(S)
    for l in range(16):
        m = jnp.minimum(m, acc[l])
    fi = jnp.where(m >= S, 0, m)

    rel0 = j0 - fi

    def in_copy(t, b):
        base = rel0 + t * R
        for q in range(R // 16):
            idxs[b][pl.ds(q * 16, 16)] = jnp.clip(
                lax.iota(jnp.int32, 16) + base + q * 16, 0, S - 1)
        return pltpu.make_async_copy(pos_hbm.at[idxs[b]], bufs[b], sins[b])

    def in_wait(b):
        pltpu.make_async_copy(pos_hbm.at[idxs[b]], bufs[b], sins[b]).wait()

    def zero_fix(t, b):
        nz = jnp.clip(fi - (j0 + t * R), 0, R)

        @pl.when(nz > 0)
        def _():
            def zrow(r, _):
                for col in range(D // 16):
                    bufs[b][r, pl.ds(col * 16, 16)] = jnp.zeros(
                        (16,), jnp.float32)
                return 0
            lax.fori_loop(0, nz, zrow, 0)

    def out_copy(t, b):
        row = pl.multiple_of(i * S + j0 + t * R, 8)
        return pltpu.make_async_copy(
            bufs[b], out_hbm.at[pl.ds(row, R), :], souts[b])

    for b in range(NBUF):
        in_copy(b, b).start()

    def group(g, _):
        for b in range(NBUF):
            t = g * NBUF + b
            in_wait(b)
            zero_fix(t, b)
            out_copy(t, b).start()
            rb = (b - LAG) % NBUF
            tr = t - LAG

            @pl.when(jnp.logical_and(tr >= 0, tr + NBUF < NCHUNK))
            def _():
                out_copy(tr, rb).wait()
                in_copy(tr + NBUF, rb).start()
        return 0

    lax.fori_loop(0, NCHUNK // NBUF, group, 0)

    for b in range(NBUF):
        out_copy(NCHUNK - NBUF + b, b).wait()


_pos_embed = functools.partial(
    pl.kernel,
    out_type=jax.ShapeDtypeStruct((B * S, D), jnp.float32),
    mesh=plsc.VectorSubcoreMesh(core_axis_name="c", subcore_axis_name="s"),
    scratch_types=(
        [pltpu.VMEM((8, S), jnp.int32)]
        + [pltpu.VMEM((R,), jnp.int32) for _ in range(NBUF)]
        + [pltpu.VMEM((R, D), jnp.float32) for _ in range(NBUF)]
        + [pltpu.SemaphoreType.DMA for _ in range(2 * NBUF)]
    ),
)(_pos_embed_body)


@jax.jit
def kernel(x, pos):
    out = _pos_embed(x.astype(jnp.int32), pos)
    return out.reshape(B, S, D)
